# baseline (speedup 1.0000x reference)
import functools

import jax
import jax.numpy as jnp
from jax import lax
from jax.experimental import pallas as pl
from jax.experimental.pallas import tpu as pltpu
from jax.experimental.pallas import tpu_sc as plsc

HIDDEN = 64
NC = 2
NS = 16
NW = NC * NS
CHUNK = 128


@functools.partial(jax.jit, static_argnums=(2,))
def _sc_gather(idx2d, table, total_rows):
    chunks_per_w = total_rows // (NW * CHUNK)
    mesh = plsc.VectorSubcoreMesh(core_axis_name="c", subcore_axis_name="s")

    @functools.partial(
        pl.kernel,
        mesh=mesh,
        out_type=jax.ShapeDtypeStruct((total_rows, HIDDEN), jnp.float32),
        scratch_types=[
            pltpu.VMEM((chunks_per_w, CHUNK), jnp.int32),
            pltpu.VMEM((CHUNK, HIDDEN), jnp.float32),
            pltpu.SemaphoreType.DMA,
        ],
        compiler_params=pltpu.CompilerParams(use_tc_tiling_on_sc=False),
    )
    def k(idx_hbm, table_hbm, out_hbm, idx_v, rows_v, sem):
        wid = lax.axis_index("s") * NC + lax.axis_index("c")
        pltpu.sync_copy(idx_hbm.at[pl.ds(wid * chunks_per_w, chunks_per_w)], idx_v)
        out_base = wid * chunks_per_w * CHUNK

        def body(g, carry):
            pltpu.async_copy(table_hbm.at[idx_v.at[g]], rows_v, sem).wait()
            pltpu.sync_copy(rows_v, out_hbm.at[pl.ds(out_base + g * CHUNK, CHUNK)])
            return carry

        lax.fori_loop(0, chunks_per_w, body, 0)

    return k(idx2d, table)


def kernel(item_seq, item_seq_len, item_embeddings):
    b, t = item_seq.shape
    total = b * t
    idx2d = item_seq.reshape(total // CHUNK, CHUNK)
    out = _sc_gather(idx2d, item_embeddings, total)
    return out.reshape(b, t, HIDDEN)


# speedup vs baseline: 1.1137x; 1.1137x over previous
import functools

import jax
import jax.numpy as jnp
from jax import lax
from jax.experimental import pallas as pl
from jax.experimental.pallas import tpu as pltpu
from jax.experimental.pallas import tpu_sc as plsc

HIDDEN = 64
NC = 2
NS = 16
NW = NC * NS
CHUNK = 128
K = 5


@functools.partial(jax.jit, static_argnums=(2,))
def _sc_gather(idx2d, table, total_rows):
    chunks_per_w = total_rows // (NW * CHUNK)
    groups = chunks_per_w // K
    assert groups % 2 == 0
    mesh = plsc.VectorSubcoreMesh(core_axis_name="c", subcore_axis_name="s")

    @functools.partial(
        pl.kernel,
        mesh=mesh,
        out_type=jax.ShapeDtypeStruct((total_rows, HIDDEN), jnp.float32),
        scratch_types=[
            pltpu.VMEM((chunks_per_w, CHUNK), jnp.int32),
            pltpu.VMEM((K * CHUNK, HIDDEN), jnp.float32),
            pltpu.VMEM((K * CHUNK, HIDDEN), jnp.float32),
            pltpu.SemaphoreType.DMA,
            pltpu.SemaphoreType.DMA,
            pltpu.SemaphoreType.DMA,
            pltpu.SemaphoreType.DMA,
        ],
        compiler_params=pltpu.CompilerParams(use_tc_tiling_on_sc=False),
    )
    def k(idx_hbm, table_hbm, out_hbm, idx_v, rows0, rows1, g0, g1, w0, w1):
        wid = lax.axis_index("s") * NC + lax.axis_index("c")
        rows = [rows0, rows1]
        gsem = [g0, g1]
        wsem = [w0, w1]
        pltpu.sync_copy(idx_hbm.at[pl.ds(wid * chunks_per_w, chunks_per_w)], idx_v)
        out_base = wid * chunks_per_w * CHUNK

        def issue_gathers(gi, b):
            for j in range(K):
                pltpu.async_copy(
                    table_hbm.at[idx_v.at[gi * K + j]],
                    rows[b].at[pl.ds(j * CHUNK, CHUNK)],
                    gsem[b],
                )

        def wait_gathers(b):
            pltpu.make_async_copy(
                table_hbm.at[pl.ds(0, K * CHUNK)], rows[b], gsem[b]
            ).wait()

        def issue_write(gi, b):
            pltpu.async_copy(
                rows[b],
                out_hbm.at[pl.ds(out_base + gi * K * CHUNK, K * CHUNK)],
                wsem[b],
            )

        def wait_write(b):
            pltpu.make_async_copy(
                rows[b], out_hbm.at[pl.ds(out_base, K * CHUNK)], wsem[b]
            ).wait()

        issue_gathers(0, 0)

        def pair_body(i, carry):
            for b in (0, 1):
                gi = 2 * i + b
                wait_gathers(b)
                issue_write(gi, b)

                @pl.when(gi >= 1)
                def _():
                    wait_write(1 - b)

                @pl.when(gi + 1 < groups)
                def _():
                    issue_gathers(gi + 1, 1 - b)

            return carry

        lax.fori_loop(0, groups // 2, pair_body, 0)
        wait_write(1)

    return k(idx2d, table)


def kernel(item_seq, item_seq_len, item_embeddings):
    b, t = item_seq.shape
    total = b * t
    idx2d = item_seq.reshape(total // CHUNK, CHUNK)
    out = _sc_gather(idx2d, item_embeddings, total)
    return out.reshape(b, t, HIDDEN)


# speedup vs baseline: 1.1718x; 1.0521x over previous
import functools

import jax
import jax.numpy as jnp
from jax import lax
from jax.experimental import pallas as pl
from jax.experimental.pallas import tpu as pltpu
from jax.experimental.pallas import tpu_sc as plsc

HIDDEN = 64
NC = 2
NS = 16
NW = NC * NS
CHUNK = 80
K = 5


@functools.partial(jax.jit, static_argnums=(2, 3))
def _sc_gather(idx2d, table2, batch, hist):
    rows_per_w = batch * hist // NW
    chunks_per_w = rows_per_w // CHUNK
    groups = chunks_per_w // K
    b_per_group = K * CHUNK // hist
    assert groups % 2 == 0 and K * CHUNK % hist == 0
    mesh = plsc.VectorSubcoreMesh(core_axis_name="c", subcore_axis_name="s")

    @functools.partial(
        pl.kernel,
        mesh=mesh,
        out_type=jax.ShapeDtypeStruct((batch, hist, HIDDEN), jnp.float32),
        scratch_types=[
            pltpu.VMEM((chunks_per_w, CHUNK), jnp.int32),
            pltpu.VMEM((K * CHUNK, HIDDEN), jnp.float32),
            pltpu.VMEM((K * CHUNK, HIDDEN), jnp.float32),
            pltpu.SemaphoreType.DMA,
            pltpu.SemaphoreType.DMA,
            pltpu.SemaphoreType.DMA,
            pltpu.SemaphoreType.DMA,
        ],
        compiler_params=pltpu.CompilerParams(use_tc_tiling_on_sc=False),
    )
    def k(idx_hbm, table_hbm, out_hbm, idx_v, rows0, rows1, g0, g1, w0, w1):
        wid = lax.axis_index("s") * NC + lax.axis_index("c")
        rows = [rows0, rows1]
        gsem = [g0, g1]
        wsem = [w0, w1]
        pltpu.sync_copy(idx_hbm.at[pl.ds(wid * chunks_per_w, chunks_per_w)], idx_v)
        out_b0 = wid * groups * b_per_group

        def issue_gathers(gi, b):
            for j in range(K):
                pltpu.async_copy(
                    table_hbm.at[idx_v.at[gi * K + j]],
                    rows[b].at[pl.ds(j * CHUNK, CHUNK)],
                    gsem[b],
                )

        def wait_gathers(b):
            pltpu.make_async_copy(
                table_hbm.at[pl.ds(0, K * CHUNK)], rows[b], gsem[b]
            ).wait()

        def issue_write(gi, b):
            for r in range(b_per_group):
                pltpu.async_copy(
                    rows[b].at[pl.ds(r * hist, hist)],
                    out_hbm.at[out_b0 + gi * b_per_group + r],
                    wsem[b],
                )

        def wait_write(b):
            for r in range(b_per_group):
                pltpu.make_async_copy(
                    rows[b].at[pl.ds(r * hist, hist)], out_hbm.at[0], wsem[b]
                ).wait()

        issue_gathers(0, 0)

        def pair_body(i, carry):
            for b in (0, 1):
                gi = 2 * i + b
                wait_gathers(b)
                issue_write(gi, b)

                @pl.when(gi >= 1)
                def _():
                    wait_write(1 - b)

                @pl.when(gi + 1 < groups)
                def _():
                    issue_gathers(gi + 1, 1 - b)

            return carry

        lax.fori_loop(0, groups // 2, pair_body, 0)
        wait_write(1)

    return k(idx2d, table2)


def kernel(item_seq, item_seq_len, item_embeddings):
    batch, hist = item_seq.shape
    n_items = item_embeddings.shape[0]
    pad_rows = (-n_items) % 8
    table2 = jnp.pad(item_embeddings, ((0, pad_rows), (0, HIDDEN))).reshape(
        2 * (n_items + pad_rows), HIDDEN
    )
    idx2d = (item_seq * 2).reshape(batch * hist // CHUNK, CHUNK)
    return _sc_gather(idx2d, table2, batch, hist)


# speedup vs baseline: 1.5896x; 1.3566x over previous
import functools

import jax
import jax.numpy as jnp
from jax import lax
from jax.experimental import pallas as pl
from jax.experimental.pallas import tpu as pltpu
from jax.experimental.pallas import tpu_sc as plsc

HIDDEN = 64
NC = 2
NS = 16
NW = NC * NS
CHUNK = 80
K = 5


@functools.partial(jax.jit, static_argnums=(2, 3))
def _sc_gather(idx2d, table2, batch, hist):
    rows_per_w = batch * hist // NW
    chunks_per_w = rows_per_w // CHUNK
    groups = chunks_per_w // K
    b_per_group = K * CHUNK // hist
    assert groups % 2 == 0 and K * CHUNK % hist == 0
    mesh = plsc.VectorSubcoreMesh(core_axis_name="c", subcore_axis_name="s")

    @functools.partial(
        pl.kernel,
        mesh=mesh,
        out_type=jax.ShapeDtypeStruct((batch, hist, 2 * HIDDEN), jnp.float32),
        scratch_types=[
            pltpu.VMEM((chunks_per_w, CHUNK), jnp.int32),
            pltpu.VMEM((K * CHUNK, HIDDEN), jnp.float32),
            pltpu.VMEM((K * CHUNK, HIDDEN), jnp.float32),
            pltpu.SemaphoreType.DMA,
            pltpu.SemaphoreType.DMA,
            pltpu.SemaphoreType.DMA,
            pltpu.SemaphoreType.DMA,
        ],
        compiler_params=pltpu.CompilerParams(use_tc_tiling_on_sc=False),
    )
    def k(idx_hbm, table_hbm, out_hbm, idx_v, rows0, rows1, g0, g1, w0, w1):
        wid = lax.axis_index("s") * NC + lax.axis_index("c")
        rows = [rows0, rows1]
        gsem = [g0, g1]
        wsem = [w0, w1]
        pltpu.sync_copy(idx_hbm.at[pl.ds(wid * chunks_per_w, chunks_per_w)], idx_v)
        out_b0 = wid * groups * b_per_group

        def issue_gathers(gi, b):
            for j in range(K):
                pltpu.async_copy(
                    table_hbm.at[idx_v.at[gi * K + j]],
                    rows[b].at[pl.ds(j * CHUNK, CHUNK)],
                    gsem[b],
                )

        def wait_gathers(b):
            pltpu.make_async_copy(
                table_hbm.at[pl.ds(0, K * CHUNK)], rows[b], gsem[b]
            ).wait()

        def issue_write(gi, b):
            for r in range(b_per_group):
                pltpu.async_copy(
                    rows[b].at[pl.ds(r * hist, hist)],
                    out_hbm.at[out_b0 + gi * b_per_group + r, :, pl.ds(0, HIDDEN)],
                    wsem[b],
                )

        def wait_write(b):
            for r in range(b_per_group):
                pltpu.make_async_copy(
                    rows[b].at[pl.ds(r * hist, hist)],
                    out_hbm.at[0, :, pl.ds(0, HIDDEN)],
                    wsem[b],
                ).wait()

        issue_gathers(0, 0)

        def pair_body(i, carry):
            for b in (0, 1):
                gi = 2 * i + b
                wait_gathers(b)
                issue_write(gi, b)

                @pl.when(gi >= 1)
                def _():
                    wait_write(1 - b)

                @pl.when(gi + 1 < groups)
                def _():
                    issue_gathers(gi + 1, 1 - b)

            return carry

        lax.fori_loop(0, groups // 2, pair_body, 0)
        wait_write(1)

    return k(idx2d, table2)


def kernel(item_seq, item_seq_len, item_embeddings):
    batch, hist = item_seq.shape
    n_items = item_embeddings.shape[0]
    pad_rows = (-n_items) % 8
    table2 = jnp.pad(item_embeddings, ((0, pad_rows), (0, HIDDEN))).reshape(
        2 * (n_items + pad_rows), HIDDEN
    )
    idx2d = (item_seq * 2).reshape(batch * hist // CHUNK, CHUNK)
    return _sc_gather(idx2d, table2, batch, hist)[:, :, :HIDDEN]
